# baseline (speedup 1.0000x reference)
import functools

import jax
import jax.numpy as jnp
from jax import lax
from jax.experimental import pallas as pl
from jax.experimental.pallas import tpu as pltpu
from jax.experimental.pallas import tpu_sc as plsc

N, E = 10000, 160000
D = 64
EP = 161280
NW = 32
PER_W = EP // NW
CH = 240
NCHUNK = PER_W // CH

NPAD = 10240
FS = 16
RHALF = EP // 2
SP_ROWS = 80896
DUMP = 80700
BLK = 1000
GRID_E = 160

_MESH = dict(core_axis_name="c", subcore_axis_name="s")
_SC_PARAMS = pltpu.CompilerParams(use_tc_tiling_on_sc=False)


def _zero_fill(zbuf):
    z = jnp.zeros((16,), jnp.float32)
    nv = zbuf.shape[1] // 16
    def row(r, _):
        for v in range(nv):
            zbuf[r, pl.ds(v * 16, 16)] = z
        return 0
    lax.fori_loop(0, zbuf.shape[0], row, 0)


def _gather2(table, idx_d, idx_s):
    mesh = plsc.VectorSubcoreMesh(**_MESH)

    @functools.partial(
        pl.kernel,
        out_type=jax.ShapeDtypeStruct((EP, 2 * D), jnp.float32),
        mesh=mesh,
        scratch_types=[
            pltpu.VMEM((PER_W,), jnp.int32),
            pltpu.VMEM((PER_W,), jnp.int32),
            pltpu.VMEM((CH, D), jnp.float32),
            pltpu.VMEM((CH, D), jnp.float32),
            pltpu.VMEM((CH, D), jnp.float32),
            pltpu.VMEM((CH, D), jnp.float32),
            pltpu.SemaphoreType.DMA,
            pltpu.SemaphoreType.DMA,
            pltpu.SemaphoreType.DMA,
            pltpu.SemaphoreType.DMA,
        ],
        compiler_params=_SC_PARAMS,
    )
    def k(tab, ind_d, ind_s, out, idx_d, idx_s, gd0, gd1, gs0, gs1,
          sg0, sg1, so0, so1):
        wid = lax.axis_index("s") * 2 + lax.axis_index("c")
        base = wid * PER_W
        pltpu.sync_copy(ind_d.at[pl.ds(base, PER_W)], idx_d)
        pltpu.sync_copy(ind_s.at[pl.ds(base, PER_W)], idx_s)
        sg = (sg0, sg1)
        so = (so0, so1)
        gbd = (gd0, gd1)
        gbs = (gs0, gs1)

        def fire(c):
            s = c % 2
            d0 = pltpu.async_copy(
                tab.at[idx_d.at[pl.ds(c * CH, CH)]], gbd[s], sg[s])
            d1 = pltpu.async_copy(
                tab.at[idx_s.at[pl.ds(c * CH, CH)]], gbs[s], sg[s])
            return d0, d1

        pend_g = fire(0)
        pend_o = [None, None]
        for c in range(NCHUNK):
            s = c % 2
            pend_g[0].wait()
            pend_g[1].wait()
            if c + 1 < NCHUNK:
                if pend_o[1 - s] is not None:
                    for d in pend_o[1 - s]:
                        d.wait()
                pend_g = fire(c + 1)
            row0 = base + c * CH
            pend_o[s] = (
                pltpu.async_copy(
                    gbd[s], out.at[pl.ds(row0, CH), pl.ds(0, D)], so[s]),
                pltpu.async_copy(
                    gbs[s], out.at[pl.ds(row0, CH), pl.ds(D, D)], so[s]),
            )
        for ds_ in pend_o:
            if ds_ is not None:
                for d in ds_:
                    d.wait()

    return k(table, idx_d, idx_s)


def _scatter_small(msgs, idx):
    mesh = plsc.VectorSubcoreMesh(**_MESH)

    @functools.partial(
        pl.kernel,
        out_type=jax.ShapeDtypeStruct((2, NPAD, 2 * D), jnp.float32),
        mesh=mesh,
        scratch_types=[
            pltpu.VMEM((PER_W,), jnp.int32),
            pltpu.VMEM((CH,), jnp.int32),
            pltpu.VMEM((CH,), jnp.int32),
            pltpu.VMEM((CH, D), jnp.float32),
            pltpu.VMEM((CH, D), jnp.float32),
            pltpu.VMEM((640, D), jnp.float32),
            pltpu.VMEM_SHARED((NPAD, D), jnp.float32),
            pltpu.SemaphoreType.DMA,
            pltpu.SemaphoreType.DMA,
            pltpu.SemaphoreType.DMA,
            pltpu.SemaphoreType.DMA,
        ],
        compiler_params=_SC_PARAMS,
    )
    def k(msg, ind, out, idxall, idxw0, idxw1, mbuf0, mbuf1, zbuf, acc,
          sm0, sm1, ss0, ss1):
        idxw = (idxw0, idxw1)
        mbuf = (mbuf0, mbuf1)
        cid = lax.axis_index("c")
        sid = lax.axis_index("s")
        wid = sid * 2 + cid
        base = wid * PER_W
        pltpu.sync_copy(ind.at[pl.ds(base, PER_W)], idxall)
        _zero_fill(zbuf)
        pltpu.sync_copy(zbuf, acc.at[pl.ds(sid * 640, 640)])
        plsc.subcore_barrier()

        sm = (sm0, sm1)
        ss = (ss0, ss1)

        def fire(c):
            s = c % 2
            return pltpu.async_copy(
                msg.at[pl.ds(base + c * CH, CH), pl.ds(0, D)], mbuf[s], sm[s])

        pend_m = fire(0)
        pend_s = [None, None]
        for c in range(NCHUNK):
            s = c % 2
            for v in range(CH // 16):
                idxw[s][pl.ds(v * 16, 16)] = idxall[pl.ds(c * CH + v * 16, 16)]
            pend_m.wait()
            if c + 1 < NCHUNK:
                if pend_s[1 - s] is not None:
                    pend_s[1 - s].wait()
                    pend_s[1 - s] = None
                pend_m = fire(c + 1)
            pend_s[s] = pltpu.async_copy(
                mbuf[s], acc.at[idxw[s]], ss[s], add=True)
        for d in pend_s:
            if d is not None:
                d.wait()
        plsc.subcore_barrier()
        pltpu.sync_copy(acc.at[pl.ds(sid * 640, 640)],
                        out.at[cid, pl.ds(sid * 640, 640), pl.ds(0, D)])

    return k(msgs, idx)


def _scatter_big(msgs, idx):
    mesh = plsc.VectorSubcoreMesh(**_MESH)
    TCH = EP // 16
    NCH2 = TCH // CH

    @functools.partial(
        pl.kernel,
        out_type=jax.ShapeDtypeStruct((EP, 2 * D), jnp.float32),
        mesh=mesh,
        scratch_types=[
            pltpu.VMEM((TCH,), jnp.int32),
            pltpu.VMEM((CH,), jnp.int32),
            pltpu.VMEM((CH,), jnp.int32),
            pltpu.VMEM((CH, FS), jnp.float32),
            pltpu.VMEM((CH, FS), jnp.float32),
            pltpu.VMEM((1024, FS), jnp.float32),
            pltpu.VMEM_SHARED((SP_ROWS, FS), jnp.float32),
            pltpu.SemaphoreType.DMA,
            pltpu.SemaphoreType.DMA,
            pltpu.SemaphoreType.DMA,
            pltpu.SemaphoreType.DMA,
        ],
        compiler_params=_SC_PARAMS,
    )
    def k(msg, ind, out, idxall, idxw0, idxw1, mbuf0, mbuf1, zbuf, acc,
          sm0, sm1, ss0, ss1):
        idxw = (idxw0, idxw1)
        mbuf = (mbuf0, mbuf1)
        cid = lax.axis_index("c")
        sid = lax.axis_index("s")
        pltpu.sync_copy(ind.at[pl.ds(sid * TCH, TCH)], idxall)
        _zero_fill(zbuf)
        sm = (sm0, sm1)
        ss = (ss0, ss1)

        for rp in range(2):
            for fq in range(4):
                @pl.when(cid == (fq % 2))
                def _phase(rp=rp, fq=fq):
                    lo = rp * RHALF
                    zd = []
                    for i in range(4):
                        zd.append(pltpu.async_copy(
                            zbuf, acc.at[pl.ds(sid * 5056 + i * 1024, 1024)],
                            sm[0]))
                    zd.append(pltpu.async_copy(
                        zbuf.at[pl.ds(0, 960)],
                        acc.at[pl.ds(sid * 5056 + 4096, 960)], sm[0]))
                    for d in zd:
                        d.wait()
                    plsc.subcore_barrier()

                    def fire_load(c, s):
                        pltpu.async_copy(
                            msg.at[pl.ds(sid * TCH + c * CH, CH),
                                   pl.ds(fq * FS, FS)],
                            mbuf[s], sm[s])

                    def wait_load(s):
                        pltpu.make_async_copy(
                            msg.at[pl.ds(0, CH), pl.ds(0, FS)],
                            mbuf[s], sm[s]).wait()

                    def wait_scat(s):
                        pltpu.make_async_copy(
                            mbuf[s], acc.at[idxw[s]], ss[s]).wait()

                    fire_load(0, 0)

                    def body(c2, carry):
                        for par in range(2):
                            c = c2 * 2 + par
                            s = par
                            for v in range(CH // 16):
                                lv = idxall[pl.ds(c * CH + v * 16, 16)]
                                il = lv - lo
                                ok = (il >= 0) & (il < RHALF)
                                idxw[s][pl.ds(v * 16, 16)] = (
                                    jnp.where(ok, il, DUMP))
                            wait_load(s)
                            @pl.when(c >= 1)
                            def _ws(s2=1 - s):
                                wait_scat(s2)
                            @pl.when(c + 1 < NCH2)
                            def _f(c=c, s2=1 - s):
                                fire_load(c + 1, s2)
                            pltpu.async_copy(
                                mbuf[s], acc.at[idxw[s]], ss[s],
                                add=True)
                        return carry
                    lax.fori_loop(0, NCH2 // 2, body, 0)
                    wait_scat(1)
                    plsc.subcore_barrier()
                    pltpu.sync_copy(
                        acc.at[pl.ds(sid * (RHALF // 16), RHALF // 16)],
                        out.at[pl.ds(lo + sid * (RHALF // 16), RHALF // 16),
                               pl.ds(fq * FS, FS)])
                    plsc.subcore_barrier()

    return k(msgs, idx)


_DN = (((1,), (0,)), ((), ()))


def _mm(a, b):
    return jax.lax.dot_general(a, b, _DN)


_mmx = _mm


def _silu(x):
    return x * jax.nn.sigmoid(x)


def _embed_kernel(x_ref, w_ref, b_ref, o_ref):
    o_ref[...] = _mm(x_ref[...], w_ref[...]) + b_ref[...]


def _tc_embed(x, w, b, rows_out, blk):
    din = x.shape[1]
    grid = (rows_out + blk - 1) // blk
    return pl.pallas_call(
        _embed_kernel,
        grid=(grid,),
        in_specs=[
            pl.BlockSpec((blk, din), lambda i: (i, 0)),
            pl.BlockSpec((din, D), lambda i: (0, 0)),
            pl.BlockSpec((1, D), lambda i: (0, 0)),
        ],
        out_specs=pl.BlockSpec((blk, D), lambda i: (i, 0)),
        out_shape=jax.ShapeDtypeStruct((rows_out, D), jnp.float32),
    )(x, w, b.reshape(1, D))


def _msg_body(g, ea, wd, ws, we, b1, w2, b2, w3, b3, wg2, bg2, o_ref):
    t = (_mmx(g[:, :D], wd[...]) + _mmx(g[:, D:], ws[...])
         + _mmx(ea, we[...]) + b1[...])
    z = _silu(t[:, :D])
    zg = _silu(t[:, D:])
    z2 = _silu(_mm(z, w2[...]) + b2[...])
    m = _mmx(z2, w3[...]) + b3[...]
    gate = jax.nn.sigmoid(_mmx(zg, wg2[...]) + bg2[...])
    o_ref[:, :D] = gate * m
    o_ref[:, D:] = jnp.zeros((o_ref.shape[0], D), jnp.float32)


def _msg_kernel_plain(g, ea, wd, ws, we, b1, w2, b2, w3, b3, wg2, bg2, o):
    _msg_body(g[...], ea[...], wd, ws, we, b1, w2, b2, w3, b3, wg2, bg2, o)


def _msg_kernel_scat(g, ea, sc, wd, ws, we, b1, w2, b2, w3, b3, wg2, bg2, o):
    _msg_body(g[...], ea[...] + sc[:, :D], wd, ws, we, b1, w2, b2, w3, b3,
              wg2, bg2, o)


def _tc_msg(g, ea, p, scat=None):
    wd = jnp.concatenate([p["node1"]["w"][:D], p["gate1"]["w"][:D]], axis=1)
    ws = jnp.concatenate([p["node1"]["w"][D:2 * D],
                          p["gate1"]["w"][D:2 * D]], axis=1)
    we = jnp.concatenate([p["node1"]["w"][2 * D:],
                          p["gate1"]["w"][2 * D:]], axis=1)
    b1 = jnp.concatenate([p["node1"]["b"], p["gate1"]["b"]]).reshape(1, 2 * D)
    espec = pl.BlockSpec((BLK, ea.shape[1]), lambda i: (i, 0))
    gspec = pl.BlockSpec((BLK, 2 * D), lambda i: (i, 0))
    wspec = lambda r, c: pl.BlockSpec((r, c), lambda i: (0, 0))
    ins = [g, ea]
    specs = [gspec, espec]
    kern = _msg_kernel_plain
    if scat is not None:
        ins.append(scat)
        specs.append(pl.BlockSpec((BLK, 2 * D), lambda i: (i, 0)))
        kern = _msg_kernel_scat
    ins += [wd, ws, we, b1, p["node2"]["w"], p["node2"]["b"].reshape(1, D),
            p["node3"]["w"], p["node3"]["b"].reshape(1, D),
            p["gate2"]["w"], p["gate2"]["b"].reshape(1, 1)]
    specs += [wspec(D, 2 * D), wspec(D, 2 * D), wspec(D, 2 * D),
              wspec(1, 2 * D), wspec(D, D), wspec(1, D), wspec(D, D),
              wspec(1, D), wspec(D, 1), wspec(1, 1)]
    return pl.pallas_call(
        kern,
        grid=(GRID_E,),
        in_specs=specs,
        out_specs=pl.BlockSpec((BLK, 2 * D), lambda i: (i, 0)),
        out_shape=jax.ShapeDtypeStruct((EP, 2 * D), jnp.float32),
    )(*ins)


def _hnew_kernel(h_ref, p_ref, o_ref):
    o_ref[...] = h_ref[...] + p_ref[0, :, :D] + p_ref[1, :, :D]


def _tc_hnew(h, parts):
    return pl.pallas_call(
        _hnew_kernel,
        grid=(10,),
        in_specs=[
            pl.BlockSpec((1000, D), lambda i: (i, 0)),
            pl.BlockSpec((2, 1000, 2 * D), lambda i: (0, i, 0)),
        ],
        out_specs=pl.BlockSpec((1000, D), lambda i: (i, 0)),
        out_shape=jax.ShapeDtypeStruct((N, D), jnp.float32),
    )(h, parts)


def _eupd_kernel(g_ref, ea_ref, w1s_ref, w1d_ref, w1e_ref, b1_ref, w2_ref,
                 b2_ref, o_ref):
    g = g_ref[...]
    ea = ea_ref[...]
    t = _silu(_mmx(g[:, D:], w1s_ref[...]) + _mmx(g[:, :D], w1d_ref[...])
              + _mmx(ea, w1e_ref[...]) + b1_ref[...])
    o_ref[...] = ea + _mm(t, w2_ref[...]) + b2_ref[...]


def _tc_eupd(g, ea, p):
    espec = pl.BlockSpec((BLK, D), lambda i: (i, 0))
    gspec = pl.BlockSpec((BLK, 2 * D), lambda i: (i, 0))
    wspec = lambda r, c: pl.BlockSpec((r, c), lambda i: (0, 0))
    w1 = p["edge1"]["w"]
    return pl.pallas_call(
        _eupd_kernel,
        grid=(GRID_E,),
        in_specs=[gspec, espec, wspec(D, D), wspec(D, D), wspec(D, D),
                  wspec(1, D), wspec(D, D), wspec(1, D)],
        out_specs=espec,
        out_shape=jax.ShapeDtypeStruct((E, D), jnp.float32),
    )(g, ea, w1[:D], w1[D:2 * D], w1[2 * D:], p["edge1"]["b"].reshape(1, D),
      p["edge2"]["w"], p["edge2"]["b"].reshape(1, D))


def _final_kernel(h_ref, p_ref, u_ref, gw_ref, gb_ref, w1_ref, b1_ref,
                  w2_ref, b2_ref, o_ref):
    hsum = jnp.sum(h_ref[...], axis=0, keepdims=True)
    psum = jnp.sum(p_ref[0, :N, :D] + p_ref[1, :N, :D], axis=0, keepdims=True)
    pool = (hsum + psum) * (1.0 / N)
    ue = _mm(u_ref[...], gw_ref[...]) + gb_ref[...]
    comb = jnp.concatenate([pool, ue], axis=1)
    z = _silu(_mm(comb, w1_ref[...]) + b1_ref[...])
    o_ref[...] = _mm(z, w2_ref[...]) + b2_ref[...]


def _tc_final(h1, parts, u2, params):
    return pl.pallas_call(
        _final_kernel,
        out_shape=jax.ShapeDtypeStruct((1, 1), jnp.float32),
    )(h1, parts, u2,
      params["global_embed"]["w"], params["global_embed"]["b"].reshape(1, D),
      params["out1"]["w"], params["out1"]["b"].reshape(1, D),
      params["out2"]["w"], params["out2"]["b"].reshape(1, 1))


def kernel(x, edge_index, edge_attr, line_graph_edge_index,
           line_graph_edge_attr, u, batch, params):
    pad0 = jnp.zeros((EP - E,), jnp.int32)
    src_g = jnp.concatenate([edge_index[0], pad0])
    dst_g = jnp.concatenate([edge_index[1], pad0])
    dst_s = jnp.concatenate([edge_index[1],
                             jnp.full((EP - E,), NPAD - 1, jnp.int32)])
    lsrc_g = jnp.concatenate([line_graph_edge_index[0], pad0])
    ldst_g = jnp.concatenate([line_graph_edge_index[1], pad0])
    ldst_s = jnp.concatenate([line_graph_edge_index[1],
                              jnp.full((EP - E,), 2 * EP, jnp.int32)])
    u2 = u.reshape(1, -1)

    l0, l1 = params["layers"][0], params["layers"][1]
    h0 = _tc_embed(x, params["node_embed"]["w"], params["node_embed"]["b"],
                   N, 2000)
    e0 = _tc_embed(edge_attr, params["edge_embed"]["w"],
                   params["edge_embed"]["b"], E, 640)

    g1 = _gather2(h0, dst_g, src_g)
    m1 = _tc_msg(g1, e0, l0["atom"])
    p1 = _scatter_small(m1, dst_s)
    h1 = _tc_hnew(h0, p1)
    g2 = _gather2(h1, dst_g, src_g)
    e1 = _tc_eupd(g2, e0, l0["atom"])

    g3 = _gather2(e1, ldst_g, lsrc_g)
    m2 = _tc_msg(g3, line_graph_edge_attr, l0["edge"])
    s2 = _scatter_big(m2, ldst_s)

    m3 = _tc_msg(g2, e1, l1["atom"], scat=s2)
    p2 = _scatter_small(m3, dst_s)

    return _tc_final(h1, p2, u2, params)
